# Initial kernel scaffold; baseline (speedup 1.0000x reference)
import jax
import jax.numpy as jnp
from jax.experimental import pallas as pl


def kernel(z, grid):
    raise NotImplementedError("write your pallas kernel here")


# speedup vs baseline: 5.5289x; 5.5289x over previous
import functools

import jax
import jax.numpy as jnp
from jax import lax
from jax.experimental import pallas as pl
from jax.experimental.pallas import tpu as pltpu
from jax.experimental.pallas import tpu_sc as plsc

N, C, IH, IW = 4, 96, 384, 384
H, W = 384, 384
HW = H * W
NPIX = N * HW
NWORKERS = 32
PPW = NPIX // NWORKERS
CHUNK = 128
NCHUNKS = PPW // CHUNK
L = 16


def _sc_sample(zt, gx, gy):
    mesh = plsc.VectorSubcoreMesh(core_axis_name="c", subcore_axis_name="s")

    @functools.partial(
        pl.kernel,
        mesh=mesh,
        compiler_params=pltpu.CompilerParams(use_tc_tiling_on_sc=False),
        out_type=jax.ShapeDtypeStruct((NPIX, C), jnp.float32),
        scratch_types=[
            pltpu.VMEM((CHUNK,), jnp.float32),
            pltpu.VMEM((CHUNK,), jnp.float32),
            pltpu.VMEM((CHUNK,), jnp.int32),
            pltpu.VMEM((CHUNK,), jnp.int32),
            pltpu.VMEM((CHUNK,), jnp.int32),
            pltpu.VMEM((CHUNK,), jnp.int32),
            pltpu.VMEM((CHUNK,), jnp.float32),
            pltpu.VMEM((CHUNK,), jnp.float32),
            pltpu.VMEM((CHUNK,), jnp.float32),
            pltpu.VMEM((CHUNK,), jnp.float32),
            pltpu.VMEM((CHUNK, C), jnp.float32),
            pltpu.VMEM((CHUNK, C), jnp.float32),
            pltpu.VMEM((CHUNK, C), jnp.float32),
            pltpu.VMEM((CHUNK, C), jnp.float32),
            pltpu.VMEM((CHUNK, C), jnp.float32),
            pltpu.SemaphoreType.DMA,
            pltpu.SemaphoreType.DMA,
            pltpu.SemaphoreType.DMA,
            pltpu.SemaphoreType.DMA,
        ],
    )
    def k(zt_hbm, gx_hbm, gy_hbm, out_hbm,
          gx_v, gy_v, inw, ine, isw, ise, wnw, wne, wsw, wse,
          rnw, rne, rsw, rse, out_v, s0, s1, s2, s3):
        wid = lax.axis_index("s") * 2 + lax.axis_index("c")
        nbase = (wid // (NWORKERS // N)) * HW
        wstart = wid * PPW

        def chunk_body(g, _):
            base = wstart + g * CHUNK
            pltpu.sync_copy(gx_hbm.at[pl.ds(base, CHUNK)], gx_v)
            pltpu.sync_copy(gy_hbm.at[pl.ds(base, CHUNK)], gy_v)

            for t in range(CHUNK // L):
                s = pl.ds(t * L, L)
                x = gx_v[s]
                y = gy_v[s]
                ix = (x + 1.0) * 0.5 * (IW - 1)
                iy = (y + 1.0) * 0.5 * (IH - 1)
                ix0 = ix.astype(jnp.int32)
                iy0 = iy.astype(jnp.int32)
                ix0f = ix0.astype(jnp.float32)
                iy0f = iy0.astype(jnp.float32)
                wx1 = (ix0f + 1.0) - ix
                wx0 = ix - ix0f
                wy1 = (iy0f + 1.0) - iy
                wy0 = iy - iy0f
                ix0c = jnp.minimum(jnp.maximum(ix0, 0), IW - 1)
                iy0c = jnp.minimum(jnp.maximum(iy0, 0), IH - 1)
                ix1c = jnp.minimum(ix0c + 1, IW - 1)
                iy1c = jnp.minimum(iy0c + 1, IH - 1)
                r0 = nbase + iy0c * IW
                r1 = nbase + iy1c * IW
                inw[s] = r0 + ix0c
                ine[s] = r0 + ix1c
                isw[s] = r1 + ix0c
                ise[s] = r1 + ix1c
                wnw[s] = wx1 * wy1
                wne[s] = wx0 * wy1
                wsw[s] = wx1 * wy0
                wse[s] = wx0 * wy0

            d0 = pltpu.async_copy(zt_hbm.at[inw], rnw, s0)
            d1 = pltpu.async_copy(zt_hbm.at[ine], rne, s1)
            d2 = pltpu.async_copy(zt_hbm.at[isw], rsw, s2)
            d3 = pltpu.async_copy(zt_hbm.at[ise], rse, s3)
            d0.wait()
            d1.wait()
            d2.wait()
            d3.wait()

            def group_body(t, _):
                s = pl.ds(t * L, L)
                av = wnw[s]
                bv = wne[s]
                cv = wsw[s]
                dv = wse[s]
                for lane in range(L):
                    i = t * L + lane
                    a = av[lane]
                    b = bv[lane]
                    c = cv[lane]
                    d = dv[lane]
                    for j in range(C // L):
                        cs = pl.ds(j * L, L)
                        out_v[i, cs] = (a * rnw[i, cs] + b * rne[i, cs]) + (
                            c * rsw[i, cs] + d * rse[i, cs])
                return 0

            lax.fori_loop(0, CHUNK // L, group_body, 0)
            pltpu.sync_copy(out_v, out_hbm.at[pl.ds(base, CHUNK)])
            return 0

        lax.fori_loop(0, NCHUNKS, chunk_body, 0)

    return k(zt, gx, gy)


def kernel(z, grid):
    zt = jnp.transpose(z, (0, 2, 3, 1)).reshape(NPIX, C)
    gx = grid[..., 0].reshape(NPIX)
    gy = grid[..., 1].reshape(NPIX)
    yt = _sc_sample(zt, gx, gy)
    return jnp.transpose(yt.reshape(N, H, W, C), (0, 3, 1, 2))
